# baseline (speedup 1.0000x reference)
import functools

import jax
import jax.numpy as jnp
from jax import lax
from jax.experimental import pallas as pl
from jax.experimental.pallas import tpu as pltpu
from jax.experimental.pallas import tpu_sc as plsc

N_NODES = 10000
N_EDGES = 640000
HIDDEN = 128
HEADS = 4
HEAD_DIM = 32
NUM_RSUS = 16
N_TYPES = 3

NC, NS = 2, 16
NW = NC * NS
NP = 10240
CB = 2048
E_P = 643072
NCHB = E_P // CB
CPT = HIDDEN // NW
NEG = -3.0e38


def _sc_body(H, src_hbm, dst_hbm, ae_hbm, asn_hbm, adn_hbm, kv_hbm, ht_hbm,
             acc_out, den_out, srcv0, dstv0, aev0, srcv1, dstv1, aev1, h_sl,
             acc, asn_sl, adn_sl, kv_v, den_acc, sem0, sem1):
    c = lax.axis_index("c")
    s = lax.axis_index("s")
    wid = c * NS + s
    tph = NW // H
    ht = wid // tph if H > 1 else wid * 0
    dslot = wid % tph
    zf = jnp.zeros((16,), jnp.float32)
    bufs = ((srcv0, dstv0, aev0, sem0), (srcv1, dstv1, aev1, sem1))

    pltpu.sync_copy(ht_hbm.at[pl.ds(wid * CPT * NP, CPT * NP)], h_sl)
    pltpu.sync_copy(asn_hbm.at[pl.ds(ht * NP, NP)], asn_sl)
    pltpu.sync_copy(adn_hbm.at[pl.ds(ht * NP, NP)], adn_sl)
    pltpu.sync_copy(kv_hbm, kv_v)

    def zacc(i, _):
        acc[pl.ds(i * 16, 16)] = zf
        return 0
    lax.fori_loop(0, CPT * NP // 16, zacc, 0)

    def zden(i, _):
        den_acc[pl.ds(i * 16, 16)] = zf
        return 0
    lax.fori_loop(0, NP // 16, zden, 0)

    kvh = kv_v[...]

    def fire(j, sv, dv, av, sem):
        eb = j * CB
        pltpu.async_copy(src_hbm.at[pl.ds(eb, CB)], sv, sem)
        pltpu.async_copy(dst_hbm.at[pl.ds(eb, CB)], dv, sem)
        pltpu.async_copy(ae_hbm.at[pl.ds(ht * E_P + eb, CB)], av, sem)

    def drain(sv, dv, av, sem):
        pltpu.make_async_copy(src_hbm.at[pl.ds(0, CB)], sv, sem).wait()
        pltpu.make_async_copy(dst_hbm.at[pl.ds(0, CB)], dv, sem).wait()
        pltpu.make_async_copy(ae_hbm.at[pl.ds(0, CB)], av, sem).wait()

    fire(0, *bufs[0])
    fire(1, *bufs[1])

    def outer(j2, _):
        for b in (0, 1):
            sv, dv, av, sem = bufs[b]
            j = j2 * 2 + b
            drain(sv, dv, av, sem)

            def edge_loop(with_den):
                @plsc.parallel_loop(0, CB // 16, 1, unroll=3)
                def _(g):
                    s16 = sv[pl.ds(g * 16, 16)]
                    d16 = dv[pl.ds(g * 16, 16)]
                    aeg = av[pl.ds(g * 16, 16)]
                    a = plsc.load_gather(asn_sl, [s16])
                    b2 = plsc.load_gather(adn_sl, [d16])
                    al = a + b2 + aeg
                    al = jnp.where(al > 0, al, 0.2 * al)
                    cb = kvh + b2
                    cdst = jnp.where(cb > 0, cb, 0.2 * cb)
                    ex = jnp.exp(al - cdst)
                    if with_den:
                        plsc.addupdate_scatter(den_acc, [d16], ex)
                    for cc in range(CPT):
                        hv = plsc.load_gather(h_sl, [cc * NP + s16])
                        plsc.addupdate_scatter(acc, [cc * NP + d16], hv * ex)

            denduty = (j % tph) == dslot

            @pl.when(denduty)
            def _():
                edge_loop(True)

            @pl.when(jnp.logical_not(denduty))
            def _():
                edge_loop(False)

            @pl.when(j + 2 < NCHB)
            def _():
                fire(j + 2, sv, dv, av, sem)
        return 0

    lax.fori_loop(0, NCHB // 2, outer, 0)

    pltpu.sync_copy(acc, acc_out.at[pl.ds(wid * CPT * NP, CPT * NP)])
    pltpu.sync_copy(den_acc, den_out.at[pl.ds(wid * NP, NP)])


def _make_sc(H):
    return pl.kernel(
        functools.partial(_sc_body, H),
        out_type=(jax.ShapeDtypeStruct((HIDDEN * NP,), jnp.float32),
                  jax.ShapeDtypeStruct((NW * NP,), jnp.float32)),
        mesh=plsc.VectorSubcoreMesh(core_axis_name="c", subcore_axis_name="s",
                                    num_cores=NC, num_subcores=NS),
        compiler_params=pltpu.CompilerParams(needs_layout_passes=False),
        scratch_types=(
            pltpu.VMEM((CB,), jnp.int32),
            pltpu.VMEM((CB,), jnp.int32),
            pltpu.VMEM((CB,), jnp.float32),
            pltpu.VMEM((CB,), jnp.int32),
            pltpu.VMEM((CB,), jnp.int32),
            pltpu.VMEM((CB,), jnp.float32),
            pltpu.VMEM((CPT * NP,), jnp.float32),
            pltpu.VMEM((CPT * NP,), jnp.float32),
            pltpu.VMEM((NP,), jnp.float32),
            pltpu.VMEM((NP,), jnp.float32),
            pltpu.VMEM((16,), jnp.float32),
            pltpu.VMEM((NP,), jnp.float32),
            pltpu.SemaphoreType.DMA,
            pltpu.SemaphoreType.DMA,
        ),
    )


_sc_h4 = _make_sc(HEADS)
_sc_h1 = _make_sc(1)


EB = E_P // 32


def _prep_ae_kernel(eat_ref, ae1wt_ref, ae2wt_ref, ae1t_ref, ae2t_ref,
                    mx_ref):
    eat = eat_ref[...]
    a1 = ae1wt_ref[...] @ eat
    a2 = ae2wt_ref[...] @ eat
    ae1t_ref[...] = a1
    ae2t_ref[...] = a2
    m1 = jnp.max(a1)
    m2 = jnp.max(a2)
    co = lax.broadcasted_iota(jnp.int32, (1, 128), 1)
    row = jnp.where(co == 0, m1, jnp.where(co == 1, m2, NEG))

    @pl.when(pl.program_id(0) == 0)
    def _():
        mx_ref[...] = jnp.full((1, 128), NEG, jnp.float32)

    mx_ref[...] = jnp.maximum(mx_ref[...], row)


def _prep_ae(eat_p, ae1wt, ae2wt):
    return pl.pallas_call(
        _prep_ae_kernel,
        grid=(32,),
        in_specs=[pl.BlockSpec((16, EB), lambda i: (0, i)),
                  pl.BlockSpec((HEADS, 16), lambda i: (0, 0)),
                  pl.BlockSpec((1, 16), lambda i: (0, 0))],
        out_specs=[pl.BlockSpec((HEADS, EB), lambda i: (0, i)),
                   pl.BlockSpec((1, EB), lambda i: (0, i)),
                   pl.BlockSpec((1, 128), lambda i: (0, 0))],
        out_shape=[jax.ShapeDtypeStruct((HEADS, E_P), jnp.float32),
                   jax.ShapeDtypeStruct((1, E_P), jnp.float32),
                   jax.ShapeDtypeStruct((1, 128), jnp.float32)],
    )(eat_p, ae1wt, ae2wt)


def _prep1_kernel(nft_ref, tyr_ref, w1at_ref, tbt_ref, ast_ref, adt_ref,
                  mx_ref, ht_ref, asnt_ref, adnt_ref, k1_ref):
    tbt = tbt_ref[...]
    ht = w1at_ref[...] @ nft_ref[...]
    ht = ht + jnp.where(tyr_ref[...] == 0, tbt[:, 0:1], tbt[:, 1:2])
    ht_ref[...] = ht
    asnt = ast_ref[...] @ ht
    asnt_ref[...] = asnt
    adnt_ref[...] = adt_ref[...] @ ht
    k1_ref[...] = jnp.full((1, 128), jnp.max(asnt) + mx_ref[0, 0], jnp.float32)


def _prep1(nft_p, tyr, w1at, tbt, ast, adt, mx):
    return pl.pallas_call(
        _prep1_kernel,
        out_shape=[jax.ShapeDtypeStruct((HIDDEN, NP), jnp.float32),
                   jax.ShapeDtypeStruct((HEADS, NP), jnp.float32),
                   jax.ShapeDtypeStruct((HEADS, NP), jnp.float32),
                   jax.ShapeDtypeStruct((1, 128), jnp.float32)],
    )(nft_p, tyr, w1at, tbt, ast, adt, mx)


def _prep2_kernel(unt_ref, dent_ref, rsel_ref, rt4_ref, b1c_ref, w2t_ref,
                  as2t_ref, ad2t_ref, mx_ref, ht2_ref, asn2t_ref, adn2t_ref,
                  k2_ref):
    dent = rsel_ref[...] @ dent_ref[...]
    d128 = rt4_ref[...] @ dent + 1e-16
    h1t = unt_ref[...] / d128 + b1c_ref[...]
    h1t = jnp.where(h1t > 0, h1t, jnp.exp(jnp.minimum(h1t, 0.0)) - 1.0)
    ht2 = w2t_ref[...] @ h1t
    ht2_ref[...] = ht2
    asn2t = as2t_ref[...] @ ht2
    asn2t_ref[...] = asn2t
    adn2t_ref[...] = ad2t_ref[...] @ ht2
    k2_ref[...] = jnp.full((1, 128), jnp.max(asn2t) + mx_ref[0, 1],
                           jnp.float32)


def _prep2(unt, dent, rsel, rt4, b1c, w2t, as2t, ad2t, mx):
    return pl.pallas_call(
        _prep2_kernel,
        out_shape=[jax.ShapeDtypeStruct((HIDDEN, NP), jnp.float32),
                   jax.ShapeDtypeStruct((1, NP), jnp.float32),
                   jax.ShapeDtypeStruct((1, NP), jnp.float32),
                   jax.ShapeDtypeStruct((1, 128), jnp.float32)],
    )(unt, dent, rsel, rt4, b1c, w2t, as2t, ad2t, mx)


def _tail_kernel(u0_ref, u1_ref, u2_ref, d0_ref, d1_ref, d2_ref, ew_ref,
                 b2c_ref, wpt_ref, bp_ref, wo1_ref, bo1c_ref, wo2_ref,
                 bo2c_ref, wo3_ref, bo3_ref, q_ref):
    xct = b2c_ref[...]
    ones = jnp.ones((1, NW), jnp.float32)
    for u_ref, d_ref, i in ((u0_ref, d0_ref, 0), (u1_ref, d1_ref, 1),
                            (u2_ref, d2_ref, 2)):
        den = ones @ d_ref[...] + 1e-16
        xct = xct + ew_ref[i, 0] * (u_ref[...] / den)
    lanes = lax.broadcasted_iota(jnp.int32, (1, NP), 1)
    scores = wpt_ref[...] @ xct + bp_ref[...]
    scores = jnp.where((lanes >= NUM_RSUS) & (lanes < N_NODES), scores, NEG)
    m = jnp.max(scores)
    e = jnp.exp(scores - m)
    w = e / jnp.sum(e)
    pool = jnp.sum(w * xct, axis=1, keepdims=True)
    rsut = xct[:, :NUM_RSUS]
    combt = jnp.concatenate(
        [rsut, jnp.broadcast_to(pool, (HIDDEN, NUM_RSUS))], axis=0)
    hq1 = jnp.maximum(
        lax.dot_general(wo1_ref[...], combt, (((0,), (0,)), ((), ())))
        + bo1c_ref[...], 0.0)
    hq2 = jnp.maximum(
        lax.dot_general(wo2_ref[...], hq1, (((0,), (0,)), ((), ())))
        + bo2c_ref[...], 0.0)
    q_ref[...] = lax.dot_general(hq2, wo3_ref[...], (((0,), (0,)), ((), ()))) \
        + bo3_ref[...]


def _tail(unts, dents, ew3, b2c, wpt, bp, Wo1, bo1c, Wo2, bo2c, Wo3, bo3r):
    return pl.pallas_call(
        _tail_kernel,
        out_shape=jax.ShapeDtypeStruct((NUM_RSUS, Wo3.shape[1]), jnp.float32),
    )(unts[0], unts[1], unts[2], dents[0], dents[1], dents[2], ew3, b2c,
      wpt, bp, Wo1, bo1c, Wo2, bo2c, Wo3, bo3r)


def kernel(node_features, node_types, edge_index, edge_attr, type_emb,
           edge_type_gates, edge_type_attention, W1, We1, att_src1, att_dst1,
           att_edge1, b1, W2, We2, att_src2, att_dst2, att_edge2, b2, Wp, bp,
           Wo1, bo1, Wo2, bo2, Wo3, bo3):
    f32 = jnp.float32
    gates = jax.nn.sigmoid(edge_type_gates)
    ew = jax.nn.softmax(edge_type_attention)
    nd = node_features.shape[1]

    nft_p = jnp.pad(node_features.T, ((0, 0), (0, NP - N_NODES)))
    tyr = jnp.pad(node_types.astype(jnp.int32), (0, NP - N_NODES))[None, :]
    eye4 = jnp.eye(HEADS, dtype=f32)
    rt4 = jnp.repeat(eye4, HEAD_DIM, axis=0)
    rsel = jnp.repeat(eye4, NW // HEADS, axis=1)

    unts, dents = [], []
    for i in range(N_TYPES):
        src_p = jnp.pad(edge_index[i, 0].astype(jnp.int32),
                        (0, E_P - N_EDGES), constant_values=NP - 1)
        dst_p = jnp.pad(edge_index[i, 1].astype(jnp.int32),
                        (0, E_P - N_EDGES), constant_values=NP - 1)
        eat_p = jnp.pad(edge_attr[i].T, ((0, 0), (0, E_P - N_EDGES)))

        ae1wt = (jnp.einsum('dhc,hc->dh',
                            We1[i].reshape(-1, HEADS, HEAD_DIM),
                            att_edge1[i]) * gates[i]).T
        ae2wt = (We2[i] @ att_edge2[i][0])[None, :] * gates[i]
        ae1t, ae2t, mx = _prep_ae(eat_p, ae1wt, ae2wt)

        ast = (att_src1[i][:, :, None] * eye4[:, None, :]).reshape(
            HIDDEN, HEADS).T
        adt = (att_dst1[i][:, :, None] * eye4[:, None, :]).reshape(
            HIDDEN, HEADS).T
        tbt = (type_emb @ W1[i][nd:]).T
        ht1, asnt, adnt, k1 = _prep1(nft_p, tyr, W1[i][:nd].T, tbt,
                                     ast, adt, mx)
        kv1 = jnp.broadcast_to(k1[0, 0], (16,))
        unt1, dent1 = _sc_h4(src_p, dst_p, ae1t.reshape(-1),
                             asnt.reshape(-1), adnt.reshape(-1), kv1,
                             ht1.reshape(-1))

        ht2, asn2t, adn2t, k2 = _prep2(unt1.reshape(HIDDEN, NP),
                                       dent1.reshape(NW, NP), rsel, rt4,
                                       b1[i][:, None], W2[i].T,
                                       att_src2[i], att_dst2[i], mx)
        kv2 = jnp.broadcast_to(k2[0, 0], (16,))
        unt2, dent2 = _sc_h1(src_p, dst_p, ae2t.reshape(-1),
                             asn2t.reshape(-1), adn2t.reshape(-1), kv2,
                             ht2.reshape(-1))
        unts.append(unt2.reshape(HIDDEN, NP))
        dents.append(dent2.reshape(NW, NP))

    b2c = (b2 * ew[:, None]).sum(0)[:, None]
    q = _tail(unts, dents, ew[:, None], b2c, Wp.T, bp[None, :], Wo1,
              bo1[:, None], Wo2, bo2[:, None], Wo3, bo3[None, :])
    return (q, edge_type_attention)


# speedup vs baseline: 1.0402x; 1.0402x over previous
import functools

import jax
import jax.numpy as jnp
from jax import lax
from jax.experimental import pallas as pl
from jax.experimental.pallas import tpu as pltpu
from jax.experimental.pallas import tpu_sc as plsc

N_NODES = 10000
N_EDGES = 640000
HIDDEN = 128
HEADS = 4
HEAD_DIM = 32
NUM_RSUS = 16
N_TYPES = 3

NC, NS = 2, 16
NW = NC * NS
NP = 10240
CB = 2048
E_P = 643072
NCHB = E_P // CB
CPT = HIDDEN // NW
NEG = -3.0e38


def _sc_body(H, src_hbm, dst_hbm, ae_hbm, asn_hbm, adn_hbm, kv_hbm, ht_hbm,
             acc_out, den_out, srcv0, dstv0, aev0, srcv1, dstv1, aev1, h_sl,
             acc, asn_sl, adn_sl, kv_v, den_acc, sem0, sem1):
    c = lax.axis_index("c")
    s = lax.axis_index("s")
    wid = c * NS + s
    tph = NW // H
    ht = wid // tph if H > 1 else wid * 0
    dslot = wid % tph
    zf = jnp.zeros((16,), jnp.float32)
    bufs = ((srcv0, dstv0, aev0, sem0), (srcv1, dstv1, aev1, sem1))

    pltpu.sync_copy(ht_hbm.at[pl.ds(wid * CPT * NP, CPT * NP)], h_sl)
    pltpu.sync_copy(asn_hbm.at[pl.ds(ht * NP, NP)], asn_sl)
    pltpu.sync_copy(adn_hbm.at[pl.ds(ht * NP, NP)], adn_sl)
    pltpu.sync_copy(kv_hbm, kv_v)

    def zacc(i, _):
        acc[pl.ds(i * 16, 16)] = zf
        return 0
    lax.fori_loop(0, CPT * NP // 16, zacc, 0)

    def zden(i, _):
        den_acc[pl.ds(i * 16, 16)] = zf
        return 0
    lax.fori_loop(0, NP // 16, zden, 0)

    kvh = kv_v[...]

    def fire(j, sv, dv, av, sem):
        eb = j * CB
        pltpu.async_copy(src_hbm.at[pl.ds(eb, CB)], sv, sem)
        pltpu.async_copy(dst_hbm.at[pl.ds(eb, CB)], dv, sem)
        pltpu.async_copy(ae_hbm.at[pl.ds(ht * E_P + eb, CB)], av, sem)

    def drain(sv, dv, av, sem):
        pltpu.make_async_copy(src_hbm.at[pl.ds(0, CB)], sv, sem).wait()
        pltpu.make_async_copy(dst_hbm.at[pl.ds(0, CB)], dv, sem).wait()
        pltpu.make_async_copy(ae_hbm.at[pl.ds(0, CB)], av, sem).wait()

    fire(0, *bufs[0])
    fire(1, *bufs[1])

    def outer(j2, _):
        for b in (0, 1):
            sv, dv, av, sem = bufs[b]
            j = j2 * 2 + b
            drain(sv, dv, av, sem)

            def edge_loop(with_den):
                @plsc.parallel_loop(0, CB // 16, 1, unroll=2)
                def _(g):
                    s16 = sv[pl.ds(g * 16, 16)]
                    d16 = dv[pl.ds(g * 16, 16)]
                    aeg = av[pl.ds(g * 16, 16)]
                    a = plsc.load_gather(asn_sl, [s16])
                    b2 = plsc.load_gather(adn_sl, [d16])
                    al = a + b2 + aeg
                    al = jnp.where(al > 0, al, 0.2 * al)
                    cb = kvh + b2
                    cdst = jnp.where(cb > 0, cb, 0.2 * cb)
                    ex = jnp.exp(al - cdst)
                    if with_den:
                        plsc.addupdate_scatter(den_acc, [d16], ex)
                    for cc in range(CPT):
                        hv = plsc.load_gather(h_sl, [cc * NP + s16])
                        plsc.addupdate_scatter(acc, [cc * NP + d16], hv * ex)

            denduty = (j % tph) == dslot

            @pl.when(denduty)
            def _():
                edge_loop(True)

            @pl.when(jnp.logical_not(denduty))
            def _():
                edge_loop(False)

            @pl.when(j + 2 < NCHB)
            def _():
                fire(j + 2, sv, dv, av, sem)
        return 0

    lax.fori_loop(0, NCHB // 2, outer, 0)

    pltpu.sync_copy(acc, acc_out.at[pl.ds(wid * CPT * NP, CPT * NP)])
    pltpu.sync_copy(den_acc, den_out.at[pl.ds(wid * NP, NP)])


def _make_sc(H):
    return pl.kernel(
        functools.partial(_sc_body, H),
        out_type=(jax.ShapeDtypeStruct((HIDDEN * NP,), jnp.float32),
                  jax.ShapeDtypeStruct((NW * NP,), jnp.float32)),
        mesh=plsc.VectorSubcoreMesh(core_axis_name="c", subcore_axis_name="s",
                                    num_cores=NC, num_subcores=NS),
        compiler_params=pltpu.CompilerParams(needs_layout_passes=False),
        scratch_types=(
            pltpu.VMEM((CB,), jnp.int32),
            pltpu.VMEM((CB,), jnp.int32),
            pltpu.VMEM((CB,), jnp.float32),
            pltpu.VMEM((CB,), jnp.int32),
            pltpu.VMEM((CB,), jnp.int32),
            pltpu.VMEM((CB,), jnp.float32),
            pltpu.VMEM((CPT * NP,), jnp.float32),
            pltpu.VMEM((CPT * NP,), jnp.float32),
            pltpu.VMEM((NP,), jnp.float32),
            pltpu.VMEM((NP,), jnp.float32),
            pltpu.VMEM((16,), jnp.float32),
            pltpu.VMEM((NP,), jnp.float32),
            pltpu.SemaphoreType.DMA,
            pltpu.SemaphoreType.DMA,
        ),
    )


_sc_h4 = _make_sc(HEADS)
_sc_h1 = _make_sc(1)


EB = E_P // 32


def _prep_ae_kernel(eat_ref, ae1wt_ref, ae2wt_ref, ae1t_ref, ae2t_ref,
                    mx_ref):
    eat = eat_ref[...]
    a1 = ae1wt_ref[...] @ eat
    a2 = ae2wt_ref[...] @ eat
    ae1t_ref[...] = a1
    ae2t_ref[...] = a2
    m1 = jnp.max(a1)
    m2 = jnp.max(a2)
    co = lax.broadcasted_iota(jnp.int32, (1, 128), 1)
    row = jnp.where(co == 0, m1, jnp.where(co == 1, m2, NEG))

    @pl.when(pl.program_id(0) == 0)
    def _():
        mx_ref[...] = jnp.full((1, 128), NEG, jnp.float32)

    mx_ref[...] = jnp.maximum(mx_ref[...], row)


def _prep_ae(eat_p, ae1wt, ae2wt):
    return pl.pallas_call(
        _prep_ae_kernel,
        grid=(32,),
        in_specs=[pl.BlockSpec((16, EB), lambda i: (0, i)),
                  pl.BlockSpec((HEADS, 16), lambda i: (0, 0)),
                  pl.BlockSpec((1, 16), lambda i: (0, 0))],
        out_specs=[pl.BlockSpec((HEADS, EB), lambda i: (0, i)),
                   pl.BlockSpec((1, EB), lambda i: (0, i)),
                   pl.BlockSpec((1, 128), lambda i: (0, 0))],
        out_shape=[jax.ShapeDtypeStruct((HEADS, E_P), jnp.float32),
                   jax.ShapeDtypeStruct((1, E_P), jnp.float32),
                   jax.ShapeDtypeStruct((1, 128), jnp.float32)],
    )(eat_p, ae1wt, ae2wt)


def _prep1_kernel(nft_ref, tyr_ref, w1at_ref, tbt_ref, ast_ref, adt_ref,
                  mx_ref, ht_ref, asnt_ref, adnt_ref, k1_ref):
    tbt = tbt_ref[...]
    ht = w1at_ref[...] @ nft_ref[...]
    ht = ht + jnp.where(tyr_ref[...] == 0, tbt[:, 0:1], tbt[:, 1:2])
    ht_ref[...] = ht
    asnt = ast_ref[...] @ ht
    asnt_ref[...] = asnt
    adnt_ref[...] = adt_ref[...] @ ht
    k1_ref[...] = jnp.full((1, 128), jnp.max(asnt) + mx_ref[0, 0], jnp.float32)


def _prep1(nft_p, tyr, w1at, tbt, ast, adt, mx):
    return pl.pallas_call(
        _prep1_kernel,
        out_shape=[jax.ShapeDtypeStruct((HIDDEN, NP), jnp.float32),
                   jax.ShapeDtypeStruct((HEADS, NP), jnp.float32),
                   jax.ShapeDtypeStruct((HEADS, NP), jnp.float32),
                   jax.ShapeDtypeStruct((1, 128), jnp.float32)],
    )(nft_p, tyr, w1at, tbt, ast, adt, mx)


def _prep2_kernel(unt_ref, dent_ref, rsel_ref, rt4_ref, b1c_ref, w2t_ref,
                  as2t_ref, ad2t_ref, mx_ref, ht2_ref, asn2t_ref, adn2t_ref,
                  k2_ref):
    dent = rsel_ref[...] @ dent_ref[...]
    d128 = rt4_ref[...] @ dent + 1e-16
    h1t = unt_ref[...] / d128 + b1c_ref[...]
    h1t = jnp.where(h1t > 0, h1t, jnp.exp(jnp.minimum(h1t, 0.0)) - 1.0)
    ht2 = w2t_ref[...] @ h1t
    ht2_ref[...] = ht2
    asn2t = as2t_ref[...] @ ht2
    asn2t_ref[...] = asn2t
    adn2t_ref[...] = ad2t_ref[...] @ ht2
    k2_ref[...] = jnp.full((1, 128), jnp.max(asn2t) + mx_ref[0, 1],
                           jnp.float32)


def _prep2(unt, dent, rsel, rt4, b1c, w2t, as2t, ad2t, mx):
    return pl.pallas_call(
        _prep2_kernel,
        out_shape=[jax.ShapeDtypeStruct((HIDDEN, NP), jnp.float32),
                   jax.ShapeDtypeStruct((1, NP), jnp.float32),
                   jax.ShapeDtypeStruct((1, NP), jnp.float32),
                   jax.ShapeDtypeStruct((1, 128), jnp.float32)],
    )(unt, dent, rsel, rt4, b1c, w2t, as2t, ad2t, mx)


def _tail_kernel(u0_ref, u1_ref, u2_ref, d0_ref, d1_ref, d2_ref, ew_ref,
                 b2c_ref, wpt_ref, bp_ref, wo1_ref, bo1c_ref, wo2_ref,
                 bo2c_ref, wo3_ref, bo3_ref, q_ref):
    xct = b2c_ref[...]
    ones = jnp.ones((1, NW), jnp.float32)
    for u_ref, d_ref, i in ((u0_ref, d0_ref, 0), (u1_ref, d1_ref, 1),
                            (u2_ref, d2_ref, 2)):
        den = ones @ d_ref[...] + 1e-16
        xct = xct + ew_ref[i, 0] * (u_ref[...] / den)
    lanes = lax.broadcasted_iota(jnp.int32, (1, NP), 1)
    scores = wpt_ref[...] @ xct + bp_ref[...]
    scores = jnp.where((lanes >= NUM_RSUS) & (lanes < N_NODES), scores, NEG)
    m = jnp.max(scores)
    e = jnp.exp(scores - m)
    w = e / jnp.sum(e)
    pool = jnp.sum(w * xct, axis=1, keepdims=True)
    rsut = xct[:, :NUM_RSUS]
    combt = jnp.concatenate(
        [rsut, jnp.broadcast_to(pool, (HIDDEN, NUM_RSUS))], axis=0)
    hq1 = jnp.maximum(
        lax.dot_general(wo1_ref[...], combt, (((0,), (0,)), ((), ())))
        + bo1c_ref[...], 0.0)
    hq2 = jnp.maximum(
        lax.dot_general(wo2_ref[...], hq1, (((0,), (0,)), ((), ())))
        + bo2c_ref[...], 0.0)
    q_ref[...] = lax.dot_general(hq2, wo3_ref[...], (((0,), (0,)), ((), ()))) \
        + bo3_ref[...]


def _tail(unts, dents, ew3, b2c, wpt, bp, Wo1, bo1c, Wo2, bo2c, Wo3, bo3r):
    return pl.pallas_call(
        _tail_kernel,
        out_shape=jax.ShapeDtypeStruct((NUM_RSUS, Wo3.shape[1]), jnp.float32),
    )(unts[0], unts[1], unts[2], dents[0], dents[1], dents[2], ew3, b2c,
      wpt, bp, Wo1, bo1c, Wo2, bo2c, Wo3, bo3r)


def kernel(node_features, node_types, edge_index, edge_attr, type_emb,
           edge_type_gates, edge_type_attention, W1, We1, att_src1, att_dst1,
           att_edge1, b1, W2, We2, att_src2, att_dst2, att_edge2, b2, Wp, bp,
           Wo1, bo1, Wo2, bo2, Wo3, bo3):
    f32 = jnp.float32
    gates = jax.nn.sigmoid(edge_type_gates)
    ew = jax.nn.softmax(edge_type_attention)
    nd = node_features.shape[1]

    nft_p = jnp.pad(node_features.T, ((0, 0), (0, NP - N_NODES)))
    tyr = jnp.pad(node_types.astype(jnp.int32), (0, NP - N_NODES))[None, :]
    eye4 = jnp.eye(HEADS, dtype=f32)
    rt4 = jnp.repeat(eye4, HEAD_DIM, axis=0)
    rsel = jnp.repeat(eye4, NW // HEADS, axis=1)

    unts, dents = [], []
    for i in range(N_TYPES):
        src_p = jnp.pad(edge_index[i, 0].astype(jnp.int32),
                        (0, E_P - N_EDGES), constant_values=NP - 1)
        dst_p = jnp.pad(edge_index[i, 1].astype(jnp.int32),
                        (0, E_P - N_EDGES), constant_values=NP - 1)
        eat_p = jnp.pad(edge_attr[i].T, ((0, 0), (0, E_P - N_EDGES)))

        ae1wt = (jnp.einsum('dhc,hc->dh',
                            We1[i].reshape(-1, HEADS, HEAD_DIM),
                            att_edge1[i]) * gates[i]).T
        ae2wt = (We2[i] @ att_edge2[i][0])[None, :] * gates[i]
        ae1t, ae2t, mx = _prep_ae(eat_p, ae1wt, ae2wt)

        ast = (att_src1[i][:, :, None] * eye4[:, None, :]).reshape(
            HIDDEN, HEADS).T
        adt = (att_dst1[i][:, :, None] * eye4[:, None, :]).reshape(
            HIDDEN, HEADS).T
        tbt = (type_emb @ W1[i][nd:]).T
        ht1, asnt, adnt, k1 = _prep1(nft_p, tyr, W1[i][:nd].T, tbt,
                                     ast, adt, mx)
        kv1 = jnp.broadcast_to(k1[0, 0], (16,))
        unt1, dent1 = _sc_h4(src_p, dst_p, ae1t.reshape(-1),
                             asnt.reshape(-1), adnt.reshape(-1), kv1,
                             ht1.reshape(-1))

        ht2, asn2t, adn2t, k2 = _prep2(unt1.reshape(HIDDEN, NP),
                                       dent1.reshape(NW, NP), rsel, rt4,
                                       b1[i][:, None], W2[i].T,
                                       att_src2[i], att_dst2[i], mx)
        kv2 = jnp.broadcast_to(k2[0, 0], (16,))
        unt2, dent2 = _sc_h1(src_p, dst_p, ae2t.reshape(-1),
                             asn2t.reshape(-1), adn2t.reshape(-1), kv2,
                             ht2.reshape(-1))
        unts.append(unt2.reshape(HIDDEN, NP))
        dents.append(dent2.reshape(NW, NP))

    b2c = (b2 * ew[:, None]).sum(0)[:, None]
    q = _tail(unts, dents, ew[:, None], b2c, Wp.T, bp[None, :], Wo1,
              bo1[:, None], Wo2, bo2[:, None], Wo3, bo3[None, :])
    return (q, edge_type_attention)
